# Initial kernel scaffold; baseline (speedup 1.0000x reference)
import jax
import jax.numpy as jnp
from jax.experimental import pallas as pl


def kernel(input_ids, emb_weight):
    raise NotImplementedError("write your pallas kernel here")


# speedup vs baseline: 3.4119x; 3.4119x over previous
import jax
import jax.numpy as jnp
from jax.experimental import pallas as pl

_SEQ_BLK = 512


def _copy_kernel(w_ref, out_ref):
    out_ref[0] = w_ref[...]


def kernel(input_ids, emb_weight):
    batch, seq_len = input_ids.shape
    dim = emb_weight.shape[1]
    blk = _SEQ_BLK
    grid = (seq_len // blk, batch)
    return pl.pallas_call(
        _copy_kernel,
        grid=grid,
        in_specs=[
            pl.BlockSpec((blk, dim), lambda i, b: (i, 0)),
        ],
        out_specs=pl.BlockSpec((1, blk, dim), lambda i, b: (b, i, 0)),
        out_shape=jax.ShapeDtypeStruct((batch, seq_len, dim), emb_weight.dtype),
    )(emb_weight)


# speedup vs baseline: 4.2200x; 1.2369x over previous
import jax
import jax.numpy as jnp
from jax.experimental import pallas as pl

_SEQ_BLK = 1024


def _copy_kernel(w_ref, out_ref):
    out_ref[0] = w_ref[...]


def kernel(input_ids, emb_weight):
    batch, seq_len = input_ids.shape
    dim = emb_weight.shape[1]
    blk = _SEQ_BLK
    grid = (seq_len // blk, batch)
    return pl.pallas_call(
        _copy_kernel,
        grid=grid,
        in_specs=[
            pl.BlockSpec((blk, dim), lambda i, b: (i, 0)),
        ],
        out_specs=pl.BlockSpec((1, blk, dim), lambda i, b: (b, i, 0)),
        out_shape=jax.ShapeDtypeStruct((batch, seq_len, dim), emb_weight.dtype),
    )(emb_weight)


# speedup vs baseline: 4.5628x; 1.0812x over previous
import jax
import jax.numpy as jnp
from jax.experimental import pallas as pl

_SEQ_BLK = 2048


def _copy_kernel(w_ref, out_ref):
    out_ref[0] = w_ref[...]


def kernel(input_ids, emb_weight):
    batch, seq_len = input_ids.shape
    dim = emb_weight.shape[1]
    blk = _SEQ_BLK
    grid = (seq_len // blk, batch)
    return pl.pallas_call(
        _copy_kernel,
        grid=grid,
        in_specs=[
            pl.BlockSpec((blk, dim), lambda i, b: (i, 0)),
        ],
        out_specs=pl.BlockSpec((1, blk, dim), lambda i, b: (b, i, 0)),
        out_shape=jax.ShapeDtypeStruct((batch, seq_len, dim), emb_weight.dtype),
    )(emb_weight)


# speedup vs baseline: 5.1339x; 1.1252x over previous
import jax
import jax.numpy as jnp
from jax.experimental import pallas as pl

_SEQ_BLK = 2048
_BATCH_BLK = 2


def _copy_kernel(w_ref, out_ref):
    out_ref[...] = jnp.broadcast_to(w_ref[...][None], out_ref.shape)


def kernel(input_ids, emb_weight):
    batch, seq_len = input_ids.shape
    dim = emb_weight.shape[1]
    blk = _SEQ_BLK
    bblk = _BATCH_BLK
    grid = (seq_len // blk, batch // bblk)
    return pl.pallas_call(
        _copy_kernel,
        grid=grid,
        in_specs=[
            pl.BlockSpec((blk, dim), lambda i, b: (i, 0)),
        ],
        out_specs=pl.BlockSpec((bblk, blk, dim), lambda i, b: (b, i, 0)),
        out_shape=jax.ShapeDtypeStruct((batch, seq_len, dim), emb_weight.dtype),
    )(emb_weight)


# speedup vs baseline: 5.1764x; 1.0083x over previous
import jax
import jax.numpy as jnp
from jax.experimental import pallas as pl

_SEQ_BLK = 1024
_BATCH_BLK = 4


def _copy_kernel(w_ref, out_ref):
    out_ref[...] = jnp.broadcast_to(w_ref[...][None], out_ref.shape)


def kernel(input_ids, emb_weight):
    batch, seq_len = input_ids.shape
    dim = emb_weight.shape[1]
    blk = _SEQ_BLK
    bblk = _BATCH_BLK
    grid = (seq_len // blk, batch // bblk)
    return pl.pallas_call(
        _copy_kernel,
        grid=grid,
        in_specs=[
            pl.BlockSpec((blk, dim), lambda i, b: (i, 0)),
        ],
        out_specs=pl.BlockSpec((bblk, blk, dim), lambda i, b: (b, i, 0)),
        out_shape=jax.ShapeDtypeStruct((batch, seq_len, dim), emb_weight.dtype),
    )(emb_weight)
